# Initial kernel scaffold; baseline (speedup 1.0000x reference)
import jax
import jax.numpy as jnp
from jax.experimental import pallas as pl


def kernel(indices, table, gamma, beta, mean, var, W1, b1, W2, b2, W3, b3):
    raise NotImplementedError("write your pallas kernel here")


# speedup vs baseline: 1.2383x; 1.2383x over previous
import functools

import jax
import jax.numpy as jnp
from jax import lax
from jax.experimental import pallas as pl
from jax.experimental.pallas import tpu as pltpu, tpu_sc as plsc

B, F, V, D = 16384, 26, 1000000, 16
SD = F * D
H1, H2, H3 = 1024, 512, 256
EPS = 1e-5

_NC, _NS = 2, 16
_NW = _NC * _NS
_N = B * F
_PER_W = _N // _NW
_NCHUNK = 8
_CH = _PER_W // _NCHUNK


def _gather_body(idx_hbm, table_hbm, out_hbm, idx_v, rows_a, rows_b, sem_in, sem_a, sem_b):
    wid = lax.axis_index("s") * _NC + lax.axis_index("c")
    base = wid * _PER_W
    pltpu.async_copy(idx_hbm.at[pl.ds(base, _PER_W)], idx_v, sem_in).wait()

    bufs = (rows_a, rows_b)
    sems = (sem_a, sem_b)

    pltpu.async_copy(table_hbm.at[idx_v.at[pl.ds(0, _CH)]], rows_a, sem_a)
    for c in range(_NCHUNK):
        cur = bufs[c % 2]
        if c + 1 < _NCHUNK:
            pltpu.async_copy(
                table_hbm.at[idx_v.at[pl.ds((c + 1) * _CH, _CH)]],
                bufs[(c + 1) % 2], sems[(c + 1) % 2])
        pltpu.make_async_copy(table_hbm.at[idx_v.at[pl.ds(c * _CH, _CH)]],
                              cur, sems[c % 2]).wait()
        pltpu.sync_copy(cur, out_hbm.at[pl.ds(base + c * _CH, _CH)])


@functools.partial(jax.jit, static_argnames=())
def _sc_gather(idx_flat, table):
    mesh = plsc.VectorSubcoreMesh(core_axis_name="c", subcore_axis_name="s")
    k = pl.kernel(
        _gather_body,
        out_type=jax.ShapeDtypeStruct((_N, D), jnp.float32),
        mesh=mesh,
        scratch_types=[
            pltpu.VMEM((_PER_W,), jnp.int32),
            pltpu.VMEM((_CH, D), jnp.float32),
            pltpu.VMEM((_CH, D), jnp.float32),
            pltpu.SemaphoreType.DMA,
            pltpu.SemaphoreType.DMA,
            pltpu.SemaphoreType.DMA,
        ],
        compiler_params=pltpu.CompilerParams(use_tc_tiling_on_sc=False),
    )
    return k(idx_flat, table)


_BLK = 512


def _mlp_body(x_ref, gamma_ref, beta_ref, mean_ref, var_ref,
              w1_ref, b1_ref, w2_ref, b2_ref, w3_ref, b3_ref, o_ref):
    x = x_ref[...]
    scale = gamma_ref[...] * lax.rsqrt(var_ref[...] + EPS)
    xn = (x - mean_ref[...]) * scale + beta_ref[...]
    h = jnp.dot(xn, w1_ref[...], preferred_element_type=jnp.float32) + b1_ref[...]
    h = jnp.maximum(h, 0.0)
    h = jnp.dot(h, w2_ref[...], preferred_element_type=jnp.float32) + b2_ref[...]
    h = jnp.maximum(h, 0.0)
    h = jnp.dot(h, w3_ref[...], preferred_element_type=jnp.float32) + b3_ref[...]
    o_ref[...] = jnp.maximum(h, 0.0)


def _mlp(x, gamma, beta, mean, var, W1, b1, W2, b2, W3, b3):
    def vspec(n):
        return pl.BlockSpec((1, n), lambda i: (0, 0))

    return pl.pallas_call(
        _mlp_body,
        grid=(B // _BLK,),
        in_specs=[
            pl.BlockSpec((_BLK, SD), lambda i: (i, 0)),
            vspec(SD), vspec(SD), vspec(SD), vspec(SD),
            pl.BlockSpec((SD, H1), lambda i: (0, 0)), vspec(H1),
            pl.BlockSpec((H1, H2), lambda i: (0, 0)), vspec(H2),
            pl.BlockSpec((H2, H3), lambda i: (0, 0)), vspec(H3),
        ],
        out_specs=pl.BlockSpec((_BLK, H3), lambda i: (i, 0)),
        out_shape=jax.ShapeDtypeStruct((B, H3), jnp.float32),
        compiler_params=pltpu.CompilerParams(
            dimension_semantics=("arbitrary",),
        ),
    )(x, gamma.reshape(1, SD), beta.reshape(1, SD), mean.reshape(1, SD),
      var.reshape(1, SD), W1, b1.reshape(1, H1), W2, b2.reshape(1, H2),
      W3, b3.reshape(1, H3))


def kernel(indices, table, gamma, beta, mean, var, W1, b1, W2, b2, W3, b3):
    idx_flat = indices.reshape(-1).astype(jnp.int32)
    rows = _sc_gather(idx_flat, table)
    x = rows.reshape(B, SD)
    return _mlp(x, gamma, beta, mean, var, W1, b1, W2, b2, W3, b3)
